# Initial kernel scaffold; baseline (speedup 1.0000x reference)
import jax
import jax.numpy as jnp
from jax.experimental import pallas as pl


def kernel(input, edge_index, edge_weight, W):
    raise NotImplementedError("write your pallas kernel here")


# speedup vs baseline: 4.5270x; 4.5270x over previous
import functools

import jax
import jax.numpy as jnp
from jax import lax
from jax.experimental import pallas as pl
from jax.experimental.pallas import tpu as pltpu
from jax.experimental.pallas import tpu_sc as plsc

NC = 2
NS = 16
NW = NC * NS
L = 16

N = 10000
NP = 10240
E = 320000
D = 128
DV = D // L
EW = E // NW
CHUNK = 80
NCHUNK = EW // CHUNK
ZROWS = 128
STRIPE = NP // NS


def _sc_spmm(x_hbm, src_hbm, dst_hbm, w_hbm, parts_hbm,
             agg_sh, src_v, dst_v, wch_v, rows_v, zbuf_v, sem):
    cid = lax.axis_index("c")
    sid = lax.axis_index("s")
    wid = cid * NS + sid

    zeros16 = jnp.zeros((L,), jnp.float32)

    def zrow(i, carry):
        for j in range(DV):
            zbuf_v[i, pl.ds(j * L, L)] = zeros16
        return carry

    lax.fori_loop(0, ZROWS, zrow, 0)
    for r in range(STRIPE // ZROWS):
        pltpu.sync_copy(zbuf_v, agg_sh.at[pl.ds(sid * STRIPE + r * ZROWS, ZROWS)])
    plsc.subcore_barrier()

    ebase = wid * EW

    def chunk_body(c, carry):
        base = ebase + c * CHUNK
        pltpu.sync_copy(src_hbm.at[pl.ds(base, CHUNK)], src_v)
        pltpu.sync_copy(dst_hbm.at[pl.ds(base, CHUNK)], dst_v)
        pltpu.sync_copy(w_hbm.at[pl.ds(base, CHUNK)], wch_v)
        pltpu.async_copy(x_hbm.at[src_v], rows_v, sem).wait()

        def scale_group(g, c2):
            wv = wch_v[pl.ds(g * L, L)]
            for k in range(L):
                wb = jnp.take_along_axis(
                    wv, jnp.full((L,), k, jnp.int32), axis=0,
                    mode="promise_in_bounds")
                e = g * L + k
                for j in range(DV):
                    sl = pl.ds(j * L, L)
                    rows_v[e, sl] = rows_v[e, sl] * wb
            return c2

        lax.fori_loop(0, CHUNK // L, scale_group, 0)
        pltpu.sync_copy(rows_v, agg_sh.at[dst_v], add=True)
        return carry

    lax.fori_loop(0, NCHUNK, chunk_body, 0)
    plsc.subcore_barrier()

    for r in range(STRIPE // ZROWS):
        sl = pl.ds(sid * STRIPE + r * ZROWS, ZROWS)
        pltpu.sync_copy(agg_sh.at[sl], parts_hbm.at[cid, sl])


_spmm_call = pl.kernel(
    _sc_spmm,
    out_type=jax.ShapeDtypeStruct((NC, NP, D), jnp.float32),
    mesh=plsc.VectorSubcoreMesh(core_axis_name="c", subcore_axis_name="s"),
    scratch_types=[
        pltpu.VMEM_SHARED((NP, D), jnp.float32),
        pltpu.VMEM((CHUNK,), jnp.int32),
        pltpu.VMEM((CHUNK,), jnp.int32),
        pltpu.VMEM((CHUNK,), jnp.float32),
        pltpu.VMEM((CHUNK, D), jnp.float32),
        pltpu.VMEM((ZROWS, D), jnp.float32),
        pltpu.SemaphoreType.DMA,
    ],
)


def _mm_body(p_ref, w_ref, o_ref):
    s = p_ref[0] + p_ref[1]
    o_ref[...] = jnp.maximum(
        jnp.dot(s, w_ref[...], preferred_element_type=jnp.float32), 0.0)


_MM_BLOCK = 1024

_mm_call = pl.pallas_call(
    _mm_body,
    grid=(NP // _MM_BLOCK,),
    in_specs=[
        pl.BlockSpec((NC, _MM_BLOCK, D), lambda i: (0, i, 0)),
        pl.BlockSpec((D, D), lambda i: (0, 0)),
    ],
    out_specs=pl.BlockSpec((_MM_BLOCK, D), lambda i: (i, 0)),
    out_shape=jax.ShapeDtypeStruct((NP, D), jnp.float32),
)


@jax.jit
def kernel(input, edge_index, edge_weight, W):
    src = edge_index[0]
    dst = edge_index[1]
    parts = _spmm_call(input, src, dst, edge_weight)
    return _mm_call(parts, W)[:N]


# speedup vs baseline: 10.6143x; 2.3447x over previous
import functools

import jax
import jax.numpy as jnp
from jax import lax
from jax.experimental import pallas as pl
from jax.experimental.pallas import tpu as pltpu
from jax.experimental.pallas import tpu_sc as plsc

NC = 2
NS = 16
NW = NC * NS
L = 16

N = 10000
NP = 10240
E = 320000
D = 128
DV = D // L
EW = E // NW
CHUNK = 80
NCHUNK = EW // CHUNK
ZROWS = 32
STRIPE = NP // NS


def _scale_rows(rows_v, w_ref):
    def scale_group(g, c2):
        wv = w_ref[pl.ds(g * L, L)]
        for k in range(L):
            wb = jnp.take_along_axis(
                wv, jnp.full((L,), k, jnp.int32), axis=0,
                mode="promise_in_bounds")
            e = g * L + k
            for j in range(DV):
                sl = pl.ds(j * L, L)
                rows_v[e, sl] = rows_v[e, sl] * wb
        return c2

    lax.fori_loop(0, CHUNK // L, scale_group, 0)


def _sc_spmm(x_hbm, src_hbm, dst_hbm, w_hbm, parts_hbm,
             agg_sh, src_v, w_a, w_b, dst_a, dst_b, rows_a, rows_b, zbuf_v,
             sem_a, sem_b, semw_a, semw_b, semd_a, semd_b):
    cid = lax.axis_index("c")
    sid = lax.axis_index("s")
    wid = cid * NS + sid

    zeros16 = jnp.zeros((L,), jnp.float32)

    def zrow(i, carry):
        for j in range(DV):
            zbuf_v[i, pl.ds(j * L, L)] = zeros16
        return carry

    lax.fori_loop(0, ZROWS, zrow, 0)
    for r in range(STRIPE // ZROWS):
        pltpu.sync_copy(zbuf_v, agg_sh.at[pl.ds(sid * STRIPE + r * ZROWS, ZROWS)])

    pltpu.sync_copy(src_hbm.at[wid], src_v)
    plsc.subcore_barrier()

    ebase = wid * EW

    def fetch(c, rows, sem, w_buf, sem_w, dst_buf, sem_d):
        pltpu.async_copy(x_hbm.at[src_v.at[c]], rows, sem)
        pltpu.async_copy(w_hbm.at[pl.ds(ebase + c * CHUNK, CHUNK)], w_buf, sem_w)
        pltpu.async_copy(
            dst_hbm.at[pl.ds(ebase + c * CHUNK, CHUNK)], dst_buf, sem_d)

    def wait(c, rows, sem, w_buf, sem_w, dst_buf, sem_d):
        pltpu.make_async_copy(x_hbm.at[src_v.at[c]], rows, sem).wait()
        pltpu.make_async_copy(
            w_hbm.at[pl.ds(ebase + c * CHUNK, CHUNK)], w_buf, sem_w).wait()
        pltpu.make_async_copy(
            dst_hbm.at[pl.ds(ebase + c * CHUNK, CHUNK)], dst_buf, sem_d).wait()

    def consume(rows, w_buf, dst_buf):
        _scale_rows(rows, w_buf)
        pltpu.sync_copy(rows, agg_sh.at[dst_buf], add=True)

    fetch(0, rows_a, sem_a, w_a, semw_a, dst_a, semd_a)

    def pipe_body(i, carry):
        c = 2 * i
        fetch(c + 1, rows_b, sem_b, w_b, semw_b, dst_b, semd_b)
        wait(c, rows_a, sem_a, w_a, semw_a, dst_a, semd_a)
        consume(rows_a, w_a, dst_a)
        fetch(c + 2, rows_a, sem_a, w_a, semw_a, dst_a, semd_a)
        wait(c + 1, rows_b, sem_b, w_b, semw_b, dst_b, semd_b)
        consume(rows_b, w_b, dst_b)
        return carry

    lax.fori_loop(0, (NCHUNK - 1) // 2, pipe_body, 0)
    wait(NCHUNK - 1, rows_a, sem_a, w_a, semw_a, dst_a, semd_a)
    consume(rows_a, w_a, dst_a)
    plsc.subcore_barrier()

    for r in range(STRIPE // ZROWS):
        sl = pl.ds(sid * STRIPE + r * ZROWS, ZROWS)
        pltpu.sync_copy(agg_sh.at[sl], parts_hbm.at[cid, sl])


_spmm_call = pl.kernel(
    _sc_spmm,
    out_type=jax.ShapeDtypeStruct((NC, NP, D), jnp.float32),
    mesh=plsc.VectorSubcoreMesh(core_axis_name="c", subcore_axis_name="s"),
    scratch_types=[
        pltpu.VMEM_SHARED((NP, D), jnp.float32),
        pltpu.VMEM((NCHUNK, CHUNK), jnp.int32),
        pltpu.VMEM((CHUNK,), jnp.float32),
        pltpu.VMEM((CHUNK,), jnp.float32),
        pltpu.VMEM((CHUNK,), jnp.int32),
        pltpu.VMEM((CHUNK,), jnp.int32),
        pltpu.VMEM((CHUNK, D), jnp.float32),
        pltpu.VMEM((CHUNK, D), jnp.float32),
        pltpu.VMEM((ZROWS, D), jnp.float32),
        pltpu.SemaphoreType.DMA,
        pltpu.SemaphoreType.DMA,
        pltpu.SemaphoreType.DMA,
        pltpu.SemaphoreType.DMA,
        pltpu.SemaphoreType.DMA,
        pltpu.SemaphoreType.DMA,
    ],
)


def _mm_body(p_ref, w_ref, o_ref):
    s = p_ref[0] + p_ref[1]
    o_ref[...] = jnp.maximum(
        jnp.dot(s, w_ref[...], preferred_element_type=jnp.float32), 0.0)


_MM_BLOCK = 1024

_mm_call = pl.pallas_call(
    _mm_body,
    grid=(NP // _MM_BLOCK,),
    in_specs=[
        pl.BlockSpec((NC, _MM_BLOCK, D), lambda i: (0, i, 0)),
        pl.BlockSpec((D, D), lambda i: (0, 0)),
    ],
    out_specs=pl.BlockSpec((_MM_BLOCK, D), lambda i: (i, 0)),
    out_shape=jax.ShapeDtypeStruct((NP, D), jnp.float32),
)


@jax.jit
def kernel(input, edge_index, edge_weight, W):
    src = edge_index[0].reshape(NW, NCHUNK, CHUNK)
    dst = edge_index[1]
    parts = _spmm_call(input, src, dst, edge_weight)
    return _mm_call(parts, W)[:N]
